# Initial kernel scaffold; baseline (speedup 1.0000x reference)
import jax
import jax.numpy as jnp
from jax.experimental import pallas as pl


def kernel(X, force):
    raise NotImplementedError("write your pallas kernel here")


# speedup vs baseline: 5.4507x; 5.4507x over previous
import functools

import jax
import jax.numpy as jnp
from jax import lax
from jax.experimental import pallas as pl
from jax.experimental.pallas import tpu as pltpu
from jax.experimental.pallas import tpu_sc as plsc

_N = 256
_DT = 0.05
_LANES = 16
_TAB_PAD = 264


def _make_body(nc, per_w, n_chunks):
    def body(x_hbm, f_hbm, o_hbm, buf, tab):
        wid = lax.axis_index("s") * nc + lax.axis_index("c")
        base = wid * per_w
        pltpu.sync_copy(f_hbm, tab)
        pltpu.sync_copy(x_hbm.at[pl.ds(base, per_w)], buf)
        lane = lax.iota(jnp.int32, _LANES)
        is_v = (lane % 2) == 1
        for i in range(n_chunks):
            w = buf[pl.ds(i * _LANES, _LANES)]
            fi = w.astype(jnp.int32)
            a = w - fi.astype(jnp.float32)
            ci = jnp.minimum(fi + 1, _N)
            f0 = plsc.load_gather(tab, [fi])
            f1 = plsc.load_gather(tab, [ci])
            stepped = w + _DT * ((1.0 - a) * f0 + a * f1)
            buf[pl.ds(i * _LANES, _LANES)] = jnp.where(is_v, stepped, w)
        pltpu.sync_copy(buf, o_hbm.at[pl.ds(base, per_w)])

    return body


@functools.lru_cache(maxsize=None)
def _build(total_words):
    info = plsc.get_sparse_core_info()
    nc, ns = info.num_cores, info.num_subcores
    nw = nc * ns
    per_w = total_words // nw
    n_chunks = per_w // _LANES
    mesh = plsc.VectorSubcoreMesh(core_axis_name="c", subcore_axis_name="s")
    return pl.kernel(
        _make_body(nc, per_w, n_chunks),
        mesh=mesh,
        out_type=jax.ShapeDtypeStruct((total_words,), jnp.float32),
        compiler_params=pltpu.CompilerParams(needs_layout_passes=False),
        scratch_types=[
            pltpu.VMEM((per_w,), jnp.float32),
            pltpu.VMEM((_TAB_PAD,), jnp.float32),
        ],
    )


def kernel(X, force):
    rows = X.shape[0]
    flat = X.reshape(-1)
    fpad = jnp.concatenate([force, jnp.zeros((_TAB_PAD - _N - 1,), force.dtype)])
    out = _build(flat.shape[0])(flat, fpad)
    return out.reshape(rows, 2)


# speedup vs baseline: 5.5809x; 1.0239x over previous
import functools

import jax
import jax.numpy as jnp
from jax import lax
from jax.experimental import pallas as pl
from jax.experimental.pallas import tpu as pltpu
from jax.experimental.pallas import tpu_sc as plsc

_N = 256
_DT = 0.05
_LANES = 16


def _make_body(nc, per_w, n_chunks):
    def body(x_hbm, f_hbm, o_hbm, buf, tab, sem_t, sem_x):
        wid = lax.axis_index("s") * nc + lax.axis_index("c")
        base = wid * per_w
        h_t = pltpu.async_copy(f_hbm, tab, sem_t)
        h_x = pltpu.async_copy(x_hbm.at[pl.ds(base, per_w)], buf, sem_x)
        h_t.wait()
        h_x.wait()
        lane = lax.iota(jnp.int32, _LANES)
        is_v = (lane % 2) == 1
        for i in range(n_chunks):
            w = buf[pl.ds(i * _LANES, _LANES)]
            fi = w.astype(jnp.int32)
            a = w - fi.astype(jnp.float32)
            ci = jnp.minimum(fi + 1, _N)
            f0 = plsc.load_gather(tab, [fi])
            f1 = plsc.load_gather(tab, [ci])
            stepped = w + _DT * ((1.0 - a) * f0 + a * f1)
            buf[pl.ds(i * _LANES, _LANES)] = jnp.where(is_v, stepped, w)
        pltpu.sync_copy(buf, o_hbm.at[pl.ds(base, per_w)])

    return body


@functools.lru_cache(maxsize=None)
def _build(total_words, tab_words):
    info = plsc.get_sparse_core_info()
    nc, ns = info.num_cores, info.num_subcores
    nw = nc * ns
    per_w = total_words // nw
    n_chunks = per_w // _LANES
    mesh = plsc.VectorSubcoreMesh(core_axis_name="c", subcore_axis_name="s")
    return pl.kernel(
        _make_body(nc, per_w, n_chunks),
        mesh=mesh,
        out_type=jax.ShapeDtypeStruct((total_words,), jnp.float32),
        compiler_params=pltpu.CompilerParams(needs_layout_passes=False),
        scratch_types=[
            pltpu.VMEM((per_w,), jnp.float32),
            pltpu.VMEM((tab_words,), jnp.float32),
            pltpu.SemaphoreType.DMA,
            pltpu.SemaphoreType.DMA,
        ],
    )


def kernel(X, force):
    rows = X.shape[0]
    flat = X.reshape(-1)
    out = _build(flat.shape[0], force.shape[0])(flat, force)
    return out.reshape(rows, 2)
